# baseline (speedup 1.0000x reference)
import functools

import jax
import jax.numpy as jnp
from jax import lax
from jax.experimental import pallas as pl
from jax.experimental.pallas import tpu as pltpu
from jax.experimental.pallas import tpu_sc as plsc

_B = 4
_S = 2048
_D = 128
_BS = _B * _S

_info = plsc.get_sparse_core_info()
_NC = _info.num_cores
_NS = _info.num_subcores
_NW = _NC * _NS
_BPW = _BS // _NW
_IDX_ROWS = _BPW // 128
_LANES = 16
_CHUNKS = _D // _LANES


_NCHUNK = 4
_CH = _BPW // _NCHUNK


def _body(x_hbm, tok_hbm, pos_hbm, out_hbm, idx_v, rows_v, pos_v, gsem, wsem):
    wid = lax.axis_index("s") * _NC + lax.axis_index("c")
    base = wid * _BPW

    pltpu.sync_copy(x_hbm.at[pl.ds(wid * _IDX_ROWS, _IDX_ROWS)], idx_v)

    ghandles = []
    for k in range(_NCHUNK):
        j, off = divmod(k * _CH, 128)
        ghandles.append(
            pltpu.async_copy(
                tok_hbm.at[idx_v.at[j, pl.ds(off, _CH)]],
                rows_v.at[pl.ds(k * _CH, _CH)],
                gsem,
            )
        )
    pos_base = (wid % (_S // _BPW)) * _BPW
    pltpu.sync_copy(pos_hbm.at[pl.ds(pos_base, _BPW)], pos_v)

    whandles = []
    for k in range(_NCHUNK):
        ghandles[k].wait()

        @plsc.parallel_loop(k * _CH, (k + 1) * _CH, unroll=2)
        def _add_row(r):
            for c in range(_CHUNKS):
                sl = pl.ds(c * _LANES, _LANES)
                plsc.addupdate(rows_v.at[r, sl], pos_v[r, sl])

        whandles.append(
            pltpu.async_copy(
                rows_v.at[pl.ds(k * _CH, _CH)],
                out_hbm.at[pl.ds(base + k * _CH, _CH)],
                wsem,
            )
        )
    for h in whandles:
        h.wait()


@jax.jit
def _embed(x_flat, token_table, pos_table):
    mesh = plsc.VectorSubcoreMesh(core_axis_name="c", subcore_axis_name="s")
    k = functools.partial(
        pl.kernel,
        mesh=mesh,
        out_type=jax.ShapeDtypeStruct((_BS, _D), jnp.float32),
        scratch_types=[
            pltpu.VMEM((_IDX_ROWS, 128), jnp.int32),
            pltpu.VMEM((_BPW, _D), jnp.float32),
            pltpu.VMEM((_BPW, _D), jnp.float32),
            pltpu.SemaphoreType.DMA,
            pltpu.SemaphoreType.DMA,
        ],
    )(_body)
    return k(x_flat, token_table, pos_table)


def kernel(x, token_table, pos_table):
    x_flat = x.reshape(_BS // 128, 128).astype(jnp.int32)
    out = _embed(x_flat, token_table, pos_table)
    return out.reshape(_B, _S, _D)


# speedup vs baseline: 1.0033x; 1.0033x over previous
import functools

import jax
import jax.numpy as jnp
from jax import lax
from jax.experimental import pallas as pl
from jax.experimental.pallas import tpu as pltpu
from jax.experimental.pallas import tpu_sc as plsc

_B = 4
_S = 2048
_D = 128
_BS = _B * _S

_info = plsc.get_sparse_core_info()
_NC = _info.num_cores
_NS = _info.num_subcores
_NW = _NC * _NS
_BPW = _BS // _NW
_IDX_ROWS = _BPW // 128
_LANES = 16
_CHUNKS = _D // _LANES


_NCHUNK = 4
_CH = _BPW // _NCHUNK


def _body(x_hbm, tok_hbm, pos_hbm, out_hbm, idx_v, rows_v, pos_v, gsem, wsem):
    wid = lax.axis_index("s") * _NC + lax.axis_index("c")
    base = wid * _BPW

    with jax.named_scope("idx_stage"):
        pltpu.sync_copy(x_hbm.at[pl.ds(wid * _IDX_ROWS, _IDX_ROWS)], idx_v)

    ghandles = []
    with jax.named_scope("gather_fire"):
        for k in range(_NCHUNK):
            j, off = divmod(k * _CH, 128)
            ghandles.append(
                pltpu.async_copy(
                    tok_hbm.at[idx_v.at[j, pl.ds(off, _CH)]],
                    rows_v.at[pl.ds(k * _CH, _CH)],
                    gsem,
                )
            )
    pos_base = (wid % (_S // _BPW)) * _BPW
    with jax.named_scope("pos_copy"):
        pltpu.sync_copy(pos_hbm.at[pl.ds(pos_base, _BPW)], pos_v)

    whandles = []
    with jax.named_scope("add_write"):
        for k in range(_NCHUNK):
            with jax.named_scope("gwait"):
                ghandles[k].wait()

            @plsc.parallel_loop(k * _CH, (k + 1) * _CH, unroll=2)
            def _add_row(r):
                for c in range(_CHUNKS):
                    sl = pl.ds(c * _LANES, _LANES)
                    plsc.addupdate(rows_v.at[r, sl], pos_v[r, sl])

            whandles.append(
                pltpu.async_copy(
                    rows_v.at[pl.ds(k * _CH, _CH)],
                    out_hbm.at[pl.ds(base + k * _CH, _CH)],
                    wsem,
                )
            )
    with jax.named_scope("drain"):
        for h in whandles:
            h.wait()


@jax.jit
def _embed(x_flat, token_table, pos_table):
    mesh = plsc.VectorSubcoreMesh(core_axis_name="c", subcore_axis_name="s")
    k = functools.partial(
        pl.kernel,
        mesh=mesh,
        out_type=jax.ShapeDtypeStruct((_BS, _D), jnp.float32),
        scratch_types=[
            pltpu.VMEM((_IDX_ROWS, 128), jnp.int32),
            pltpu.VMEM((_BPW, _D), jnp.float32),
            pltpu.VMEM((_BPW, _D), jnp.float32),
            pltpu.SemaphoreType.DMA,
            pltpu.SemaphoreType.DMA,
        ],
    )(_body)
    return k(x_flat, token_table, pos_table)


def kernel(x, token_table, pos_table):
    x_flat = x.reshape(_BS // 128, 128).astype(jnp.int32)
    out = _embed(x_flat, token_table, pos_table)
    return out.reshape(_B, _S, _D)


# speedup vs baseline: 1.0937x; 1.0900x over previous
import functools

import jax
import jax.numpy as jnp
from jax import lax
from jax.experimental import pallas as pl
from jax.experimental.pallas import tpu as pltpu
from jax.experimental.pallas import tpu_sc as plsc

_B = 4
_S = 2048
_D = 128

_info = plsc.get_sparse_core_info()
_NC = _info.num_cores
_NS = _info.num_subcores
_NW = _NC * _NS
_SPW = _S // _NW
_LANES = 16
_CHUNKS = _D // _LANES


def _body(x_hbm, tok_hbm, pos_hbm, out_hbm, idx_v, rows_v, pos_v,
          psem, isem, gsem, wsem):
    wid = lax.axis_index("s") * _NC + lax.axis_index("c")
    s0 = wid * _SPW

    ph = pltpu.async_copy(pos_hbm.at[pl.ds(s0, _SPW)], pos_v, psem)

    ihandles = [
        pltpu.async_copy(x_hbm.at[b, pl.ds(s0, _SPW)], idx_v.at[b], isem)
        for b in range(_B)
    ]
    for h in ihandles:
        h.wait()

    ghandles = [
        pltpu.async_copy(tok_hbm.at[idx_v.at[b]], rows_v.at[b], gsem)
        for b in range(_B)
    ]
    ph.wait()

    whandles = []
    for b in range(_B):
        ghandles[b].wait()

        @plsc.parallel_loop(0, _SPW, unroll=2)
        def _add_row(r):
            for c in range(_CHUNKS):
                sl = pl.ds(c * _LANES, _LANES)
                plsc.addupdate(rows_v.at[b, r, sl], pos_v[r, sl])

        whandles.append(
            pltpu.async_copy(rows_v.at[b], out_hbm.at[b, pl.ds(s0, _SPW)], wsem)
        )
    for h in whandles:
        h.wait()


@jax.jit
def _embed(x, token_table, pos_table):
    mesh = plsc.VectorSubcoreMesh(core_axis_name="c", subcore_axis_name="s")
    k = functools.partial(
        pl.kernel,
        mesh=mesh,
        out_type=jax.ShapeDtypeStruct((_B, _S, _D), jnp.float32),
        scratch_types=[
            pltpu.VMEM((_B, _SPW), jnp.int32),
            pltpu.VMEM((_B, _SPW, _D), jnp.float32),
            pltpu.VMEM((_SPW, _D), jnp.float32),
            pltpu.SemaphoreType.DMA,
            pltpu.SemaphoreType.DMA,
            pltpu.SemaphoreType.DMA,
            pltpu.SemaphoreType.DMA,
        ],
    )(_body)
    return k(x, token_table, pos_table)


def kernel(x, token_table, pos_table):
    return _embed(x.astype(jnp.int32), token_table, pos_table)


# speedup vs baseline: 1.0953x; 1.0015x over previous
import functools

import jax
import jax.numpy as jnp
from jax import lax
from jax.experimental import pallas as pl
from jax.experimental.pallas import tpu as pltpu
from jax.experimental.pallas import tpu_sc as plsc

_B = 4
_S = 2048
_D = 128

_info = plsc.get_sparse_core_info()
_NC = _info.num_cores
_NS = _info.num_subcores
_NW = _NC * _NS
_SPW = _S // _NW
_LANES = 16
_CHUNKS = _D // _LANES


def _body(x_hbm, tok_hbm, pos_hbm, out_hbm, idx_v, rows_v, pos_v,
          psem, isem, gsem, wsem):
    wid = lax.axis_index("s") * _NC + lax.axis_index("c")
    s0 = wid * _SPW

    ph = pltpu.async_copy(pos_hbm.at[pl.ds(s0, _SPW)], pos_v, psem)

    with jax.named_scope("idx_stage"):
        ihandles = [
            pltpu.async_copy(x_hbm.at[b, pl.ds(s0, _SPW)], idx_v.at[b], isem)
            for b in range(_B)
        ]
        for h in ihandles:
            h.wait()

    with jax.named_scope("gather_fire"):
        ghandles = [
            pltpu.async_copy(tok_hbm.at[idx_v.at[b]], rows_v.at[b], gsem)
            for b in range(_B)
        ]
    ph.wait()

    whandles = []
    for g in range(_B // 2):
        with jax.named_scope("gwait"):
            ghandles[g * 2].wait()
            ghandles[g * 2 + 1].wait()

        with jax.named_scope("add"):

            @plsc.parallel_loop(0, _SPW, unroll=2)
            def _add_row(r):
                for c in range(_CHUNKS):
                    sl = pl.ds(c * _LANES, _LANES)
                    p = pos_v[r, sl]
                    plsc.addupdate(rows_v.at[g * 2, r, sl], p)
                    plsc.addupdate(rows_v.at[g * 2 + 1, r, sl], p)

        whandles.append(
            pltpu.async_copy(
                rows_v.at[pl.ds(g * 2, 2)],
                out_hbm.at[pl.ds(g * 2, 2), pl.ds(s0, _SPW)],
                wsem,
            )
        )
    with jax.named_scope("drain"):
        for h in whandles:
            h.wait()


@jax.jit
def _embed(x, token_table, pos_table):
    mesh = plsc.VectorSubcoreMesh(core_axis_name="c", subcore_axis_name="s")
    k = functools.partial(
        pl.kernel,
        mesh=mesh,
        out_type=jax.ShapeDtypeStruct((_B, _S, _D), jnp.float32),
        scratch_types=[
            pltpu.VMEM((_B, _SPW), jnp.int32),
            pltpu.VMEM((_B, _SPW, _D), jnp.float32),
            pltpu.VMEM((_SPW, _D), jnp.float32),
            pltpu.SemaphoreType.DMA,
            pltpu.SemaphoreType.DMA,
            pltpu.SemaphoreType.DMA,
            pltpu.SemaphoreType.DMA,
        ],
    )(_body)
    return k(x, token_table, pos_table)


def kernel(x, token_table, pos_table):
    return _embed(x.astype(jnp.int32), token_table, pos_table)
